# Initial kernel scaffold; baseline (speedup 1.0000x reference)
import jax
import jax.numpy as jnp
from jax.experimental import pallas as pl


def kernel(x_hex, ei_flat, ea_flat, lengths, We, W1, gamma, beta, W2):
    raise NotImplementedError("write your pallas kernel here")


# speedup vs baseline: 1.8956x; 1.8956x over previous
import functools

import jax
import jax.numpy as jnp
from jax.experimental import pallas as pl
from jax.experimental.pallas import tpu as pltpu

N = 10000
E = 320000
D = 128
ED = 16

_NB = 10
_BN = N // _NB


def _k1_body(x_ref, p0_ref, p1_ref, w0_ref, w1_ref, h0_ref, h1_ref, s_ref):
    i = pl.program_id(0)

    @pl.when(i == 0)
    def _():
        s_ref[...] = jnp.zeros_like(s_ref)

    x = x_ref[...]
    stats = []
    for p_ref, w_ref, h_ref in ((p0_ref, w0_ref, h0_ref), (p1_ref, w1_ref, h1_ref)):
        p = p_ref[...]
        out = p[:, :D] / (p[:, D:] + 1e-16) + x
        h = jnp.dot(out, w_ref[...], preferred_element_type=jnp.float32)
        h_ref[...] = h
        stats.append(jnp.sum(h, axis=0, keepdims=True))
        stats.append(jnp.sum(h * h, axis=0, keepdims=True))
    s_ref[...] += jnp.concatenate(stats + [jnp.zeros((4, 2 * D), jnp.float32)], axis=0)


def _k2_body(h0_ref, h1_ref, s_ref, gb_ref, w0_ref, w1_ref, o_ref, *, leaky):
    s = s_ref[...]
    acc = None
    for ci, (h_ref, w_ref) in enumerate(((h0_ref, w0_ref), (h1_ref, w1_ref))):
        h = h_ref[...]
        mean = s[2 * ci:2 * ci + 1, :] / N
        var = s[2 * ci + 1:2 * ci + 2, :] / N - mean * mean
        g = gb_ref[2 * ci:2 * ci + 1, :]
        b = gb_ref[2 * ci + 1:2 * ci + 2, :]
        hn = (h - mean) * (g * jax.lax.rsqrt(var + 1e-5)) + b
        hn = jnp.maximum(hn, 0.0)
        y = jnp.dot(hn, w_ref[...], preferred_element_type=jnp.float32)
        acc = y if acc is None else acc + y
    if leaky:
        acc = jnp.where(acc > 0, acc, 0.01 * acc)
    o_ref[...] = acc


def _node_stage(x, p0, p1, w1t0, w1t1, gb, w2t0, w2t1, leaky):
    h0, h1, s = pl.pallas_call(
        _k1_body,
        grid=(_NB,),
        in_specs=[
            pl.BlockSpec((_BN, D), lambda i: (i, 0)),
            pl.BlockSpec((_BN, 2 * D), lambda i: (i, 0)),
            pl.BlockSpec((_BN, 2 * D), lambda i: (i, 0)),
            pl.BlockSpec((D, 2 * D), lambda i: (0, 0)),
            pl.BlockSpec((D, 2 * D), lambda i: (0, 0)),
        ],
        out_specs=[
            pl.BlockSpec((_BN, 2 * D), lambda i: (i, 0)),
            pl.BlockSpec((_BN, 2 * D), lambda i: (i, 0)),
            pl.BlockSpec((8, 2 * D), lambda i: (0, 0)),
        ],
        out_shape=[
            jax.ShapeDtypeStruct((N, 2 * D), jnp.float32),
            jax.ShapeDtypeStruct((N, 2 * D), jnp.float32),
            jax.ShapeDtypeStruct((8, 2 * D), jnp.float32),
        ],
    )(x, p0, p1, w1t0, w1t1)
    return pl.pallas_call(
        functools.partial(_k2_body, leaky=leaky),
        grid=(_NB,),
        in_specs=[
            pl.BlockSpec((_BN, 2 * D), lambda i: (i, 0)),
            pl.BlockSpec((_BN, 2 * D), lambda i: (i, 0)),
            pl.BlockSpec((8, 2 * D), lambda i: (0, 0)),
            pl.BlockSpec((8, 2 * D), lambda i: (0, 0)),
            pl.BlockSpec((2 * D, D), lambda i: (0, 0)),
            pl.BlockSpec((2 * D, D), lambda i: (0, 0)),
        ],
        out_specs=pl.BlockSpec((_BN, D), lambda i: (i, 0)),
        out_shape=jax.ShapeDtypeStruct((N, D), jnp.float32),
    )(h0, h1, s, gb, w2t0, w2t1)


def _edge_pass(x, src, dst, ea, We_c, e0, e1):
    idx = jnp.arange(E, dtype=jnp.int32)
    mask = (idx >= e0) & (idx < e1)
    x_j = jnp.take(x, src, axis=0)
    msg = jax.nn.relu(x_j + ea @ We_c.T) + 1e-7
    ex = jnp.where(mask[:, None], jnp.exp(msg), 0.0)
    payload = jnp.concatenate([msg * ex, ex], axis=1)
    return jax.ops.segment_sum(payload, dst, num_segments=N)


def kernel(x_hex, ei_flat, ea_flat, lengths, We, W1, gamma, beta, W2):
    src, dst = ei_flat[0], ei_flat[1]
    W1t = jnp.transpose(W1, (0, 2, 1))
    W2t = jnp.transpose(W2, (0, 2, 1))
    x = x_hex
    for i in range(2):
        c0, c1 = 2 * i, 2 * i + 1
        e_mid = lengths[0]
        e_end = lengths[0] + lengths[1]
        p0 = _edge_pass(x, src, dst, ea_flat, We[c0], 0, e_mid)
        p1 = _edge_pass(x, src, dst, ea_flat, We[c1], e_mid, e_end)
        gb = jnp.concatenate([
            gamma[c0:c0 + 1], beta[c0:c0 + 1], gamma[c1:c1 + 1], beta[c1:c1 + 1],
            jnp.zeros((4, 2 * D), jnp.float32)], axis=0)
        x = _node_stage(x, p0, p1, W1t[c0], W1t[c1], gb, W2t[c0], W2t[c1],
                        leaky=(i < 1))
    return x


# speedup vs baseline: 4.6320x; 2.4436x over previous
import functools

import jax
import jax.numpy as jnp
from jax import lax
from jax.experimental import pallas as pl
from jax.experimental.pallas import tpu as pltpu
from jax.experimental.pallas import tpu_sc as plsc

N = 10000
E = 320000
D = 128
ED = 16
DH = D // 2

_NB = 10
_BN = N // _NB

_NT = 16
_CH = 128
_RPT = 624
_REM = N - _RPT * _NT
_REMB = _RPT * _NT
_NCHT = 79

_EB = 1000
_NEB = E // _EB


def _emb_body(lens_ref, ea_ref, w_ref, o_ref):
    b = pl.program_id(1)
    len0 = lens_ref[0]
    ea = ea_ref[...]
    y0 = jnp.dot(ea, w_ref[0, 0], preferred_element_type=jnp.float32)
    y1 = jnp.dot(ea, w_ref[0, 1], preferred_element_type=jnp.float32)
    rows = b * _EB + lax.broadcasted_iota(jnp.int32, (_EB, 1), 0)
    o_ref[...] = jnp.where(rows < len0, y0, y1)[None]


def _emb_stage(lens, ea, wq):
    return pl.pallas_call(
        _emb_body,
        grid=(2, _NEB),
        in_specs=[
            pl.BlockSpec(memory_space=pltpu.SMEM),
            pl.BlockSpec((_EB, ED), lambda h, b: (b, 0)),
            pl.BlockSpec((1, 2, ED, DH), lambda h, b: (h, 0, 0, 0)),
        ],
        out_specs=pl.BlockSpec((1, _EB, DH), lambda h, b: (h, b, 0)),
        out_shape=jax.ShapeDtypeStruct((2, E, DH), jnp.float32),
    )(lens, ea, wq)


def _edge_body(xh, emb, src, dst, bnd, zeros, out,
               acc, srcb, dstb, gidx, embb, xrb, pay, bndv):
    k = lax.axis_index("c")
    t = lax.axis_index("s")
    pltpu.sync_copy(bnd, bndv)
    bv = bndv[...]

    def sget(i):
        return bv[i]

    rows0 = t * _RPT
    xoff = k * N

    for cc in range(2):
        e0 = sget(3 * cc)
        e1 = sget(3 * cc + 1)
        a8 = sget(3 * cc + 2)
        pltpu.sync_copy(zeros, acc.at[pl.ds(rows0, _RPT)])

        @pl.when(t == 0)
        def _():
            pltpu.sync_copy(zeros.at[pl.ds(0, _REM)],
                            acc.at[pl.ds(_REMB, _REM)])
        plsc.subcore_barrier()

        def chunk(i, carry):
            eb = pl.multiple_of(a8 + (i * _NT + t) * _CH, 8)

            @pl.when(eb < e1)
            def _():
                pltpu.sync_copy(src.at[pl.ds(eb, _CH)], srcb)
                pltpu.sync_copy(dst.at[pl.ds(eb, _CH)], dstb)
                pltpu.sync_copy(emb.at[pl.ds(k * E + eb, _CH)], embb)

                def fixidx(j, c2):
                    gidx[pl.ds(j * 16, 16)] = srcb[pl.ds(j * 16, 16)] + xoff
                    return c2
                lax.fori_loop(0, _CH // 16, fixidx, 0)
                pltpu.sync_copy(xh.at[gidx], xrb)

                def rowloop(r, c2):
                    ge = eb + r
                    w = jnp.where((ge >= e0) & (ge < e1),
                                  jnp.float32(1.0), jnp.float32(0.0))
                    for q in range(DH // 16):
                        xv = xrb[r, pl.ds(q * 16, 16)]
                        ev = embb[r, pl.ds(q * 16, 16)]
                        m = jnp.maximum(xv + ev, 0.0) + 1e-7
                        ex = jnp.exp(m) * w
                        pay[r, pl.ds(q * 16, 16)] = m * ex
                        pay[r, pl.ds(DH + q * 16, 16)] = ex
                    return c2
                lax.fori_loop(0, _CH, rowloop, 0)
                pltpu.sync_copy(pay, acc.at[dstb], add=True)
            return carry

        lax.fori_loop(0, _NCHT, chunk, 0)
        plsc.subcore_barrier()
        obase = cc * 2 * N + k * N
        pltpu.sync_copy(acc.at[pl.ds(rows0, _RPT)],
                        out.at[pl.ds(obase + rows0, _RPT)])

        @pl.when(t == 0)
        def _():
            pltpu.sync_copy(acc.at[pl.ds(_REMB, _REM)],
                            out.at[pl.ds(obase + _REMB, _REM)])


def _edge_stage(xh, emb, src, dst, bnd, zeros):
    mesh = plsc.VectorSubcoreMesh(core_axis_name="c", subcore_axis_name="s")
    f = functools.partial(
        pl.kernel, _edge_body, mesh=mesh,
        compiler_params=pltpu.CompilerParams(use_tc_tiling_on_sc=False),
        out_type=jax.ShapeDtypeStruct((4 * N, D), jnp.float32),
        scratch_types=[
            pltpu.VMEM_SHARED((N, D), jnp.float32),
            pltpu.VMEM((_CH,), jnp.int32),
            pltpu.VMEM((_CH,), jnp.int32),
            pltpu.VMEM((_CH,), jnp.int32),
            pltpu.VMEM((_CH, DH), jnp.float32),
            pltpu.VMEM((_CH, DH), jnp.float32),
            pltpu.VMEM((_CH, D), jnp.float32),
            pltpu.VMEM((16,), jnp.int32),
        ],
    )()
    return f(xh, emb, src, dst, bnd, zeros)


def _k1_body(x_ref, p0l_ref, p0h_ref, p1l_ref, p1h_ref, w0_ref, w1_ref,
             h0_ref, h1_ref, s_ref):
    i = pl.program_id(0)

    @pl.when(i == 0)
    def _():
        s_ref[...] = jnp.zeros_like(s_ref)

    x = x_ref[...]
    stats = []
    for pl_ref, ph_ref, w_ref, h_ref in (
            (p0l_ref, p0h_ref, w0_ref, h0_ref),
            (p1l_ref, p1h_ref, w1_ref, h1_ref)):
        plo = pl_ref[...]
        phi = ph_ref[...]
        agg = jnp.concatenate(
            [plo[:, :DH] / (plo[:, DH:] + 1e-16),
             phi[:, :DH] / (phi[:, DH:] + 1e-16)], axis=1)
        h = jnp.dot(agg + x, w_ref[...], preferred_element_type=jnp.float32)
        h_ref[...] = h
        stats.append(jnp.sum(h, axis=0, keepdims=True))
        stats.append(jnp.sum(h * h, axis=0, keepdims=True))
    s_ref[...] += jnp.concatenate(
        stats + [jnp.zeros((4, 2 * D), jnp.float32)], axis=0)


def _k2_body(h0_ref, h1_ref, s_ref, gb_ref, w0_ref, w1_ref, o_ref, oh_ref,
             *, leaky):
    s = s_ref[...]
    acc = None
    for ci, (h_ref, w_ref) in enumerate(((h0_ref, w0_ref), (h1_ref, w1_ref))):
        h = h_ref[...]
        mean = s[2 * ci:2 * ci + 1, :] / N
        var = s[2 * ci + 1:2 * ci + 2, :] / N - mean * mean
        g = gb_ref[2 * ci:2 * ci + 1, :]
        b = gb_ref[2 * ci + 1:2 * ci + 2, :]
        hn = (h - mean) * (g * lax.rsqrt(var + 1e-5)) + b
        hn = jnp.maximum(hn, 0.0)
        y = jnp.dot(hn, w_ref[...], preferred_element_type=jnp.float32)
        acc = y if acc is None else acc + y
    if leaky:
        acc = jnp.where(acc > 0, acc, 0.01 * acc)
    o_ref[...] = acc
    oh_ref[0] = acc[:, :DH]
    oh_ref[1] = acc[:, DH:]


def _node_stage(x, p0l, p0h, p1l, p1h, w1t0, w1t1, gb, w2t0, w2t1, leaky):
    h0, h1, s = pl.pallas_call(
        _k1_body,
        grid=(_NB,),
        in_specs=[
            pl.BlockSpec((_BN, D), lambda i: (i, 0)),
            pl.BlockSpec((_BN, D), lambda i: (i, 0)),
            pl.BlockSpec((_BN, D), lambda i: (i, 0)),
            pl.BlockSpec((_BN, D), lambda i: (i, 0)),
            pl.BlockSpec((_BN, D), lambda i: (i, 0)),
            pl.BlockSpec((D, 2 * D), lambda i: (0, 0)),
            pl.BlockSpec((D, 2 * D), lambda i: (0, 0)),
        ],
        out_specs=[
            pl.BlockSpec((_BN, 2 * D), lambda i: (i, 0)),
            pl.BlockSpec((_BN, 2 * D), lambda i: (i, 0)),
            pl.BlockSpec((8, 2 * D), lambda i: (0, 0)),
        ],
        out_shape=[
            jax.ShapeDtypeStruct((N, 2 * D), jnp.float32),
            jax.ShapeDtypeStruct((N, 2 * D), jnp.float32),
            jax.ShapeDtypeStruct((8, 2 * D), jnp.float32),
        ],
    )(x, p0l, p0h, p1l, p1h, w1t0, w1t1)
    return pl.pallas_call(
        functools.partial(_k2_body, leaky=leaky),
        grid=(_NB,),
        in_specs=[
            pl.BlockSpec((_BN, 2 * D), lambda i: (i, 0)),
            pl.BlockSpec((_BN, 2 * D), lambda i: (i, 0)),
            pl.BlockSpec((8, 2 * D), lambda i: (0, 0)),
            pl.BlockSpec((8, 2 * D), lambda i: (0, 0)),
            pl.BlockSpec((2 * D, D), lambda i: (0, 0)),
            pl.BlockSpec((2 * D, D), lambda i: (0, 0)),
        ],
        out_specs=[
            pl.BlockSpec((_BN, D), lambda i: (i, 0)),
            pl.BlockSpec((2, _BN, DH), lambda i: (0, i, 0)),
        ],
        out_shape=[
            jax.ShapeDtypeStruct((N, D), jnp.float32),
            jax.ShapeDtypeStruct((2, N, DH), jnp.float32),
        ],
    )(h0, h1, s, gb, w2t0, w2t1)


def kernel(x_hex, ei_flat, ea_flat, lengths, We, W1, gamma, beta, W2):
    src = ei_flat[0]
    dst = ei_flat[1]
    W1t = jnp.transpose(W1, (0, 2, 1))
    W2t = jnp.transpose(W2, (0, 2, 1))

    len0 = lengths[0].astype(jnp.int32)
    len1 = lengths[1].astype(jnp.int32)
    e1_1 = len0 + len1
    a8_1 = (len0 // 8) * 8
    zero = jnp.zeros((), jnp.int32)
    bnd = jnp.stack([zero, len0, zero, len0, e1_1, a8_1,
                     zero, zero, zero, zero, zero, zero,
                     zero, zero, zero, zero]).astype(jnp.int32)
    lens_smem = lengths[:1].astype(jnp.int32)
    zeros_init = jnp.zeros((_RPT, D), jnp.float32)

    x = x_hex
    xh = jnp.concatenate([x_hex[:, :DH], x_hex[:, DH:]], axis=0)
    for i in range(2):
        c0, c1 = 2 * i, 2 * i + 1
        w0t, w1t = We[c0].T, We[c1].T
        wq = jnp.stack([jnp.stack([w0t[:, :DH], w1t[:, :DH]]),
                        jnp.stack([w0t[:, DH:], w1t[:, DH:]])])
        emb = _emb_stage(lens_smem, ea_flat, wq)
        p = _edge_stage(xh, emb.reshape(2 * E, DH), src, dst, bnd, zeros_init)
        p = p.reshape(2, 2, N, D)
        gb = jnp.concatenate([
            gamma[c0:c0 + 1], beta[c0:c0 + 1], gamma[c1:c1 + 1],
            beta[c1:c1 + 1], jnp.zeros((4, 2 * D), jnp.float32)], axis=0)
        x, xh2 = _node_stage(x, p[0, 0], p[0, 1], p[1, 0], p[1, 1],
                             W1t[c0], W1t[c1], gb, W2t[c0], W2t[c1],
                             leaky=(i < 1))
        xh = xh2.reshape(2 * N, DH)
    return x
